# baseline (speedup 1.0000x reference)
import jax
import jax.numpy as jnp
from jax import lax
from jax.experimental import pallas as pl
from jax.experimental.pallas import tpu as pltpu

_TOPK = 100


def _topk_body(x_ref, o_ref, c_ref):
    _NEG = float("-inf")
    _BIG = 2**30
    f32 = jnp.float32
    hm = x_ref[0, :80, :, :]
    m = jnp.max(hm, axis=2)
    a_io = lax.broadcasted_iota(jnp.int32, (80, 128), 0)
    b_io = lax.broadcasted_iota(jnp.int32, (80, 128), 1)
    ridx = a_io * 128 + b_io
    lane1 = lax.broadcasted_iota(jnp.int32, (1, 128), 1)

    def p2_step(k, carry):
        m, rsel, msel = carry
        gmax = jnp.max(m)
        rstar = jnp.min(jnp.where(m == gmax, ridx, _BIG))
        m = jnp.where(ridx == rstar, _NEG, m)
        km = lane1 == k
        rsel = jnp.where(km, rstar, rsel)
        msel = jnp.where(km, gmax, msel)
        return (m, rsel, msel)

    rsel0 = jnp.zeros((1, 128), jnp.int32)
    msel0 = jnp.full((1, 128), _NEG, f32)
    _, rsel, msel = lax.fori_loop(0, _TOPK, p2_step, (m, rsel0, msel0))

    def gat_step(k, carry):
        r = jnp.sum(jnp.where(lane1 == k, rsel, 0))
        c_ref[pl.ds(k, 1), :] = x_ref[0, r >> 7, pl.ds(r & 127, 1), :]
        return carry

    lax.fori_loop(0, _TOPK, gat_step, 0)

    def p3_step(k, carry):
        mc, oi, os = carry
        gmax = jnp.max(mc)
        r = jnp.min(jnp.where(mc == gmax, rsel, _BIG))
        pstar = jnp.min(jnp.where((mc == gmax) & (rsel == r), lane1, _BIG))
        row = c_ref[pl.ds(pstar, 1), :]
        lstar = jnp.min(jnp.where(row == gmax, lane1, _BIG))
        newrow = jnp.where(lane1 == lstar, _NEG, row)
        c_ref[pl.ds(pstar, 1), :] = newrow
        mc = jnp.where(lane1 == pstar, jnp.max(newrow), mc)
        km = lane1 == k
        oi = jnp.where(km, r * 128 + lstar, oi)
        os = jnp.where(km, gmax, os)
        return (mc, oi, os)

    zi = jnp.zeros((1, 128), jnp.int32)
    zs = jnp.zeros((1, 128), f32)
    _, idx, score = lax.fori_loop(0, _TOPK, p3_step, (msel, zi, zs))

    y = (idx >> 7) & 127
    xl = idx & 127
    spat = idx & 16383
    sub2d = lax.broadcasted_iota(jnp.int32, (128, 128), 0)
    by = (sub2d == y).astype(f32)
    bx = (sub2d == xl).astype(f32)
    dn = (((0,), (0,)), ((), ()))

    def gather_ch(ch):
        p = lax.dot_general(
            ch, by, dn, preferred_element_type=f32, precision=lax.Precision.HIGHEST
        )
        return jnp.sum(p * bx, axis=0, keepdims=True)

    bw = gather_ch(x_ref[0, 80, :, :])
    bh = gather_ch(x_ref[0, 81, :, :])
    xo = gather_ch(x_ref[0, 82, :, :])
    yo = gather_ch(x_ref[0, 83, :, :])
    cls = idx.astype(f32) / f32(16384.0)
    cy = spat.astype(f32) / f32(128.0) + yo
    cx = xl.astype(f32) + xo
    hw = 0.5 * bw
    hh = 0.5 * bh
    s4 = f32(4.0)
    o_ref[0] = jnp.concatenate(
        [(cx - hw) * s4, (cy - hh) * s4, (cx + hw) * s4, (cy + hh) * s4, cls, score],
        axis=0,
    )


def _build(interpret=False):
    return pl.pallas_call(
        _topk_body,
        grid=(16,),
        in_specs=[pl.BlockSpec((1, 84, 128, 128), lambda b: (b, 0, 0, 0))],
        out_specs=pl.BlockSpec((1, 6, 128), lambda b: (b, 0, 0)),
        out_shape=jax.ShapeDtypeStruct((16, 6, 128), jnp.float32),
        scratch_shapes=[pltpu.VMEM((128, 128), jnp.float32)],
        interpret=interpret,
    )


@jax.jit
def kernel(x):
    rows = _build()(x)
    return jnp.transpose(rows, (0, 2, 1))[:, :_TOPK, :]


# speedup vs baseline: 2.2174x; 2.2174x over previous
import jax
import jax.numpy as jnp
from jax import lax
from jax.experimental import pallas as pl
from jax.experimental.pallas import tpu as pltpu

_TOPK = 100
_B = 4


def _topk_body(x_ref, o_ref):
    _NEG = float("-inf")
    _BIG = 2**30
    f32 = jnp.float32
    a_io = lax.broadcasted_iota(jnp.int32, (80, 128), 0)
    b_io = lax.broadcasted_iota(jnp.int32, (80, 128), 1)
    ridx = a_io * 128 + b_io
    lane1 = lax.broadcasted_iota(jnp.int32, (1, 128), 1)

    ms = tuple(jnp.max(x_ref[i, :80, :, :], axis=2) for i in range(_B))

    def step(k, carry):
        ms, ois, oss = carry
        km = lane1 == k
        nm, noi, nos = [], [], []
        for i in range(_B):
            m = ms[i]
            gmax = jnp.max(m)
            rstar = jnp.min(jnp.where(m == gmax, ridx, _BIG))
            c = rstar >> 7
            y = rstar & 127
            row = x_ref[i, c, pl.ds(y, 1), :]
            lstar = jnp.min(jnp.where(row == gmax, lane1, _BIG))
            newrow = jnp.where(lane1 == lstar, _NEG, row)
            x_ref[i, c, pl.ds(y, 1), :] = newrow
            nm.append(jnp.where(ridx == rstar, jnp.max(newrow), m))
            noi.append(jnp.where(km, rstar * 128 + lstar, ois[i]))
            nos.append(jnp.where(km, gmax, oss[i]))
        return (tuple(nm), tuple(noi), tuple(nos))

    zi = tuple(jnp.zeros((1, 128), jnp.int32) for _ in range(_B))
    zs = tuple(jnp.zeros((1, 128), f32) for _ in range(_B))
    _, idxs, scores = lax.fori_loop(0, _TOPK, step, (ms, zi, zs))

    sub2d = lax.broadcasted_iota(jnp.int32, (128, 128), 0)
    dn = (((0,), (0,)), ((), ()))
    for i in range(_B):
        idx = idxs[i]
        y = (idx >> 7) & 127
        xl = idx & 127
        spat = idx & 16383
        by = (sub2d == y).astype(f32)
        bx = (sub2d == xl).astype(f32)

        def gather_ch(ch):
            p = lax.dot_general(
                ch, by, dn, preferred_element_type=f32,
                precision=lax.Precision.HIGHEST,
            )
            return jnp.sum(p * bx, axis=0, keepdims=True)

        bw = gather_ch(x_ref[i, 80, :, :])
        bh = gather_ch(x_ref[i, 81, :, :])
        xo = gather_ch(x_ref[i, 82, :, :])
        yo = gather_ch(x_ref[i, 83, :, :])
        cls = idx.astype(f32) / f32(16384.0)
        cy = spat.astype(f32) / f32(128.0) + yo
        cx = xl.astype(f32) + xo
        hw = 0.5 * bw
        hh = 0.5 * bh
        s4 = f32(4.0)
        o_ref[i] = jnp.concatenate(
            [(cx - hw) * s4, (cy - hh) * s4, (cx + hw) * s4, (cy + hh) * s4,
             cls, scores[i]],
            axis=0,
        )


def _build(interpret=False):
    return pl.pallas_call(
        _topk_body,
        grid=(16 // _B,),
        in_specs=[pl.BlockSpec((_B, 84, 128, 128), lambda b: (b, 0, 0, 0))],
        out_specs=pl.BlockSpec((_B, 6, 128), lambda b: (b, 0, 0)),
        out_shape=jax.ShapeDtypeStruct((16, 6, 128), jnp.float32),
        interpret=interpret,
    )


@jax.jit
def kernel(x):
    rows = _build()(x)
    return jnp.transpose(rows, (0, 2, 1))[:, :_TOPK, :]
